# baseline (speedup 1.0000x reference)
import jax
import jax.numpy as jnp
from jax.experimental import pallas as pl
from jax.experimental.pallas import tpu as pltpu


def _make_body(tile_s, n_s, S):
    inv_s = 1.0 / S

    def _body(x_ref, w1_ref, b1_ref, w2_ref, b2_ref, o_ref, acc_ref):
        s = pl.program_id(1)
        x = x_ref[0]
        part = jnp.sum(x, axis=0, keepdims=True, dtype=jnp.float32)

        @pl.when(s == 0)
        def _():
            acc_ref[...] = part

        @pl.when(s != 0)
        def _():
            acc_ref[...] += part

        o_ref[0, pl.ds(s * tile_s, tile_s), :] = x

        @pl.when(s == n_s - 1)
        def _():
            pooled = acc_ref[...] * inv_s
            h = jnp.maximum(
                jnp.dot(pooled, w1_ref[...],
                        preferred_element_type=jnp.float32) + b1_ref[...],
                0.0)
            gate = jax.nn.sigmoid(
                jnp.dot(h, w2_ref[...],
                        preferred_element_type=jnp.float32) + b2_ref[...])
            o_ref[0] = (o_ref[0] * gate.astype(o_ref.dtype))

    return _body


def kernel(x, w1, b1, w2, b2):
    N, C, D, H, W = x.shape
    S = D * H * W
    Ch = w1.shape[1]

    tile_s = 1024
    while S % tile_s:
        tile_s //= 2
    tile_s = min(tile_s, S)
    n_s = S // tile_s

    x_nsc = jnp.transpose(x, (0, 2, 3, 4, 1)).reshape(N, S, C)
    b1r = b1.reshape(1, Ch).astype(jnp.float32)
    b2r = b2.reshape(1, C).astype(jnp.float32)

    out = pl.pallas_call(
        _make_body(tile_s, n_s, S),
        out_shape=jax.ShapeDtypeStruct((N, S, C), x.dtype),
        grid=(N, n_s),
        in_specs=[
            pl.BlockSpec((1, tile_s, C), lambda n, s: (n, s, 0)),
            pl.BlockSpec((C, Ch), lambda n, s: (0, 0)),
            pl.BlockSpec((1, Ch), lambda n, s: (0, 0)),
            pl.BlockSpec((Ch, C), lambda n, s: (0, 0)),
            pl.BlockSpec((1, C), lambda n, s: (0, 0)),
        ],
        out_specs=pl.BlockSpec((1, S, C), lambda n, s: (n, 0, 0)),
        scratch_shapes=[pltpu.VMEM((1, C), jnp.float32)],
        compiler_params=pltpu.CompilerParams(
            dimension_semantics=("parallel", "arbitrary"),
            vmem_limit_bytes=48 * 1024 * 1024),
    )(x_nsc, w1.astype(jnp.float32), b1r, w2.astype(jnp.float32), b2r)

    return out.reshape(N, D, H, W, C).transpose(0, 4, 1, 2, 3)


# speedup vs baseline: 1.3308x; 1.3308x over previous
import jax
import jax.numpy as jnp
from jax.experimental import pallas as pl
from jax.experimental.pallas import tpu as pltpu


def _fused_body(x_ref, w1_ref, b1_ref, w2_ref, b2_ref, o_ref):
    x = x_ref[0]
    inv_s = jnp.float32(1.0 / x.shape[0])
    pooled = jnp.sum(x, axis=0, keepdims=True,
                     dtype=jnp.float32) * inv_s
    h = jnp.maximum(
        jnp.dot(pooled, w1_ref[...],
                preferred_element_type=jnp.float32) + b1_ref[...], 0.0)
    gate = jax.nn.sigmoid(
        jnp.dot(h, w2_ref[...],
                preferred_element_type=jnp.float32) + b2_ref[...])
    o_ref[0] = (x * gate.astype(x.dtype)).astype(o_ref.dtype)


def kernel(x, w1, b1, w2, b2):
    N, C, D, H, W = x.shape
    S = D * H * W
    Ch = w1.shape[1]

    x_nsc = jnp.transpose(x, (0, 2, 3, 4, 1)).reshape(N, S, C)
    b1r = b1.reshape(1, Ch).astype(jnp.float32)
    b2r = b2.reshape(1, C).astype(jnp.float32)

    out = pl.pallas_call(
        _fused_body,
        out_shape=jax.ShapeDtypeStruct((N, S, C), x.dtype),
        grid=(N,),
        in_specs=[
            pl.BlockSpec((1, S, C), lambda n: (n, 0, 0)),
            pl.BlockSpec((C, Ch), lambda n: (0, 0)),
            pl.BlockSpec((1, Ch), lambda n: (0, 0)),
            pl.BlockSpec((Ch, C), lambda n: (0, 0)),
            pl.BlockSpec((1, C), lambda n: (0, 0)),
        ],
        out_specs=pl.BlockSpec((1, S, C), lambda n: (n, 0, 0)),
        compiler_params=pltpu.CompilerParams(
            dimension_semantics=("parallel",),
            vmem_limit_bytes=48 * 1024 * 1024),
    )(x_nsc, w1.astype(jnp.float32), b1r, w2.astype(jnp.float32), b2r)

    return out.reshape(N, D, H, W, C).transpose(0, 4, 1, 2, 3)
